# baseline (speedup 1.0000x reference)
import functools

import jax
import jax.numpy as jnp
from jax import lax
from jax.experimental import pallas as pl
from jax.experimental.pallas import tpu as pltpu
from jax.experimental.pallas import tpu_sc as plsc

B = 16384
D = 32
NW = 32
BPW = B // NW
CH = BPW // 128
BLK = 2048
OFFS = (0, 20, 33, 65, 89, 96)


def _sc_gather_body(utab, btab, uidx2, bidx2, ue_out, be_out,
                    uiv, uev, biv, bev, usem, bsem):
    wid = lax.axis_index("s") * 2 + lax.axis_index("c")
    rbase = wid * CH
    base = wid * BPW
    pltpu.sync_copy(uidx2.at[pl.ds(rbase, CH)], uiv)
    pltpu.sync_copy(bidx2.at[pl.ds(rbase, CH)], biv)
    ucopies = [pltpu.async_copy(utab.at[uiv.at[c]], uev.at[pl.ds(c * 128, 128)], usem)
               for c in range(CH)]
    bcopies = [pltpu.async_copy(btab.at[biv.at[c]], bev.at[pl.ds(c * 128, 128)], bsem)
               for c in range(CH)]
    for c in ucopies:
        c.wait()
    pltpu.sync_copy(uev, ue_out.at[pl.ds(base, BPW)])
    for c in bcopies:
        c.wait()
    pltpu.sync_copy(bev, be_out.at[pl.ds(base, BPW)])


def _sc_gather(utab, btab, uidx2, bidx2):
    mesh = plsc.VectorSubcoreMesh(core_axis_name="c", subcore_axis_name="s")
    return pl.kernel(
        _sc_gather_body,
        mesh=mesh,
        compiler_params=pltpu.CompilerParams(use_tc_tiling_on_sc=False),
        out_type=[jax.ShapeDtypeStruct((B, D), jnp.float32),
                  jax.ShapeDtypeStruct((B, D), jnp.float32)],
        scratch_types=[
            pltpu.VMEM((CH, 128), jnp.int32),
            pltpu.VMEM((BPW, D), jnp.float32),
            pltpu.VMEM((CH, 128), jnp.int32),
            pltpu.VMEM((BPW, D), jnp.float32),
            pltpu.SemaphoreType.DMA,
            pltpu.SemaphoreType.DMA,
        ],
    )(utab, btab, uidx2, bidx2)


def _tc_body(ut_ref, bt_ref, ue_ref, be_ref, tf_ref,
             wut_ref, wbt_ref, wpack_ref, ttab_ref, out_ref):
    f32 = jnp.float32

    def dot_t(a, b):
        return lax.dot_general(a, b, (((1,), (1,)), ((), ())),
                               precision=lax.Precision.HIGHEST,
                               preferred_element_type=f32)

    up = dot_t(ut_ref[...], wut_ref[...]) + wpack_ref[7:8, :]
    bp = dot_t(bt_ref[...], wbt_ref[...]) + wpack_ref[8:9, :]
    ue = ue_ref[...]
    be = be_ref[...]
    wm1 = wpack_ref[0:1, :]
    wm2 = wpack_ref[1:2, :]
    wu1 = wpack_ref[2:3, :]
    wu2 = wpack_ref[3:4, :]
    wb1 = wpack_ref[4:5, :]
    wb2 = wpack_ref[5:6, :]
    wt = wpack_ref[6:7, :]
    t = (ue * (be * wm1 + wu1) + up * (bp * wm2 + wu2)
         + be * wb1 + bp * wb2)
    s = dot_t(t, jnp.ones((1, D), f32))

    tvec = dot_t(wt, ttab_ref[...])
    lanes = lax.broadcasted_iota(jnp.int32, (BLK, 128), 1)
    acc = jnp.zeros((BLK, 128), f32)
    for j, off in enumerate(OFFS):
        col = tf_ref[:, j:j + 1] + off
        acc = acc + (lanes == col).astype(f32)
    ts = dot_t(acc, tvec)

    out_ref[...] = s + ts + wpack_ref[9:10, 0:1]


def _tc_combine(ut, bt, ue, be, tf, wut, wbt, wpack, ttab):
    grid = B // BLK
    return pl.pallas_call(
        _tc_body,
        grid=(grid,),
        in_specs=[
            pl.BlockSpec((BLK, 128), lambda i: (i, 0)),
            pl.BlockSpec((BLK, 128), lambda i: (i, 0)),
            pl.BlockSpec((BLK, D), lambda i: (i, 0)),
            pl.BlockSpec((BLK, D), lambda i: (i, 0)),
            pl.BlockSpec((BLK, 6), lambda i: (i, 0)),
            pl.BlockSpec((D, 128), lambda i: (0, 0)),
            pl.BlockSpec((D, 128), lambda i: (0, 0)),
            pl.BlockSpec((16, D), lambda i: (0, 0)),
            pl.BlockSpec((128, D), lambda i: (0, 0)),
        ],
        out_specs=pl.BlockSpec((BLK, 1), lambda i: (i, 0)),
        out_shape=jax.ShapeDtypeStruct((B, 1), jnp.float32),
    )(ut, bt, ue, be, tf, wut, wbt, wpack, ttab)


def kernel(user, book, user_tag_embedding, book_tag_embedding, time_features,
           user_table, book_table, W_ut, b_ut, W_bt, b_bt,
           year_t, month_t, day_t, hour_t, weekday_t, isweekend_t,
           W_out, b_out):
    uidx2 = user.astype(jnp.int32).reshape(B // 128, 128)
    bidx2 = book.astype(jnp.int32).reshape(B // 128, 128)
    ue, be = _sc_gather(user_table, book_table, uidx2, bidx2)

    w = W_out.reshape(224)
    wpack = jnp.zeros((16, D), jnp.float32)
    for r in range(7):
        wpack = wpack.at[r].set(w[r * 32:(r + 1) * 32])
    wpack = wpack.at[7].set(b_ut)
    wpack = wpack.at[8].set(b_bt)
    wpack = wpack.at[9, 0].set(b_out[0])

    ttab = jnp.zeros((128, D), jnp.float32)
    ttab = ttab.at[0:20, 0:10].set(year_t)
    ttab = ttab.at[20:33, 10:15].set(month_t)
    ttab = ttab.at[33:65, 15:20].set(day_t)
    ttab = ttab.at[65:89, 20:25].set(hour_t)
    ttab = ttab.at[89:96, 25:30].set(weekday_t)
    ttab = ttab.at[96:98, 30:32].set(isweekend_t)

    tf = time_features.astype(jnp.int32)
    out = _tc_combine(user_tag_embedding, book_tag_embedding, ue, be, tf,
                      W_ut, W_bt, wpack, ttab)
    return out.reshape(B)
